# baseline (speedup 1.0000x reference)
import functools

import jax
import jax.numpy as jnp
from jax import lax
from jax.experimental import pallas as pl
from jax.experimental.pallas import tpu as pltpu
from jax.experimental.pallas import tpu_sc as plsc

BATCH = 16384
EMBED = 64
FEAT = 128
HID = 128
NROWS = 1000000

NC = 2
NS = 16
NW = NC * NS
BPW = BATCH // NW


def _gather_body(uix, pix, ut1, pt1, u_out, p_out, idx_v, rows_v, sem):
    wid = lax.axis_index("s") * NC + lax.axis_index("c")
    base = wid * BPW

    for tab, ix, out in ((ut1, uix, u_out), (pt1, pix, p_out)):
        pltpu.sync_copy(ix.at[pl.ds(base, BPW)], idx_v)

        def group(g):
            v = idx_v[pl.ds(g * 16, 16)]
            for lane in range(16):
                i = g * 16 + lane
                pltpu.async_copy(
                    tab.at[pl.ds(v[lane] * EMBED, EMBED)],
                    rows_v.at[pl.ds(i * EMBED, EMBED)], sem)

        pl.loop(0, BPW // 16)(group)
        pltpu.make_async_copy(
            tab.at[pl.ds(0, BPW * EMBED)], rows_v, sem).wait()
        pltpu.sync_copy(rows_v, out.at[pl.ds(base * EMBED, BPW * EMBED)])


def _sc_gather(uix, pix, ut1, pt1):
    mesh = plsc.VectorSubcoreMesh(core_axis_name="c", subcore_axis_name="s")
    fn = functools.partial(
        pl.kernel,
        mesh=mesh,
        out_type=(
            jax.ShapeDtypeStruct((BATCH * EMBED,), jnp.float32),
            jax.ShapeDtypeStruct((BATCH * EMBED,), jnp.float32),
        ),
        scratch_types=[
            pltpu.VMEM((BPW,), jnp.int32),
            pltpu.VMEM((BPW * EMBED,), jnp.float32),
            pltpu.SemaphoreType.DMA,
        ],
    )(_gather_body)
    return fn(uix, pix, ut1, pt1)


def _mlp_body(u, p, f, w1u, w1p, w1f, b1, w2, b2, w3t, b3, o):
    x1 = jnp.dot(u[:], w1u[:], preferred_element_type=jnp.float32)
    x1 = x1 + jnp.dot(p[:], w1p[:], preferred_element_type=jnp.float32)
    x1 = x1 + jnp.dot(f[:], w1f[:], preferred_element_type=jnp.float32)
    h1 = jnp.maximum(x1 + b1[:], 0.0)
    h2 = jnp.maximum(
        jnp.dot(h1, w2[:], preferred_element_type=jnp.float32) + b2[:], 0.0)
    s = jnp.sum(h2 * w3t[:], axis=1, keepdims=True) + b3[:]
    o[:] = 1.0 / (1.0 + jnp.exp(-s))


def _tc_mlp(u, p, f, w1u, w1p, w1f, b1, w2, b2, w3t, b3, tile=512):
    grid = BATCH // tile
    full = lambda i: (0, 0)
    return pl.pallas_call(
        _mlp_body,
        grid=(grid,),
        in_specs=[
            pl.BlockSpec((tile, EMBED), lambda i: (i, 0)),
            pl.BlockSpec((tile, EMBED), lambda i: (i, 0)),
            pl.BlockSpec((tile, FEAT), lambda i: (i, 0)),
            pl.BlockSpec((EMBED, HID), full),
            pl.BlockSpec((EMBED, HID), full),
            pl.BlockSpec((FEAT, HID), full),
            pl.BlockSpec((1, HID), full),
            pl.BlockSpec((HID, HID), full),
            pl.BlockSpec((1, HID), full),
            pl.BlockSpec((1, HID), full),
            pl.BlockSpec((1, 1), full),
        ],
        out_specs=pl.BlockSpec((tile, 1), lambda i: (i, 0)),
        out_shape=jax.ShapeDtypeStruct((BATCH, 1), jnp.float32),
    )(u, p, f, w1u, w1p, w1f, b1, w2, b2, w3t, b3)


def kernel(user_indices, post_indices, features, user_table, post_table,
           W1, b1, W2, b2, W3, b3):
    ui = user_indices.astype(jnp.int32)
    pi = post_indices.astype(jnp.int32)
    u1, p1 = _sc_gather(ui, pi,
                        user_table.reshape(NROWS * EMBED),
                        post_table.reshape(NROWS * EMBED))
    o = _tc_mlp(
        u1.reshape(BATCH, EMBED), p1.reshape(BATCH, EMBED), features,
        W1[:EMBED], W1[EMBED:2 * EMBED], W1[2 * EMBED:],
        b1.reshape(1, HID), W2, b2.reshape(1, HID),
        W3.reshape(1, HID), b3.reshape(1, 1))
    return o.reshape(BATCH)


# speedup vs baseline: 1.0050x; 1.0050x over previous
import functools

import jax
import jax.numpy as jnp
from jax import lax
from jax.experimental import pallas as pl
from jax.experimental.pallas import tpu as pltpu
from jax.experimental.pallas import tpu_sc as plsc

BATCH = 16384
EMBED = 64
FEAT = 128
HID = 128
NROWS = 1000000

NC = 2
NS = 16
NW = NC * NS
BPW = BATCH // NW


def _gather_body(uix, pix, ut2, pt1, u_out, p_out,
                 uix_v, pix_v, urows_v, prows_v, sem):
    wid = lax.axis_index("s") * NC + lax.axis_index("c")
    base = wid * BPW
    pltpu.sync_copy(uix.at[pl.ds(base, BPW)], uix_v)
    pltpu.sync_copy(pix.at[pl.ds(base, BPW)], pix_v)

    def group(g):
        vu = uix_v[pl.ds(g * 16, 16)]
        vq = lax.shift_right_logical(vu, 1)
        vp = pix_v[pl.ds(g * 16, 16)]
        for lane in range(16):
            i = g * 16 + lane
            pltpu.async_copy(
                ut2.at[pl.ds(vq[lane], 1)],
                urows_v.at[pl.ds(i, 1)], sem)
            pltpu.async_copy(
                pt1.at[pl.ds(vp[lane] * EMBED, EMBED)],
                prows_v.at[pl.ds(i * EMBED, EMBED)], sem)

    pl.loop(0, BPW // 16)(group)
    pltpu.make_async_copy(
        ut2.at[pl.ds(0, BPW)], urows_v, sem).wait()
    pltpu.make_async_copy(
        pt1.at[pl.ds(0, BPW * EMBED)], prows_v, sem).wait()
    pltpu.sync_copy(urows_v, u_out.at[pl.ds(base, BPW)])
    pltpu.sync_copy(prows_v, p_out.at[pl.ds(base * EMBED, BPW * EMBED)])


def _sc_gather(uix, pix, ut2, pt1):
    mesh = plsc.VectorSubcoreMesh(core_axis_name="c", subcore_axis_name="s")
    fn = functools.partial(
        pl.kernel,
        mesh=mesh,
        out_type=(
            jax.ShapeDtypeStruct((BATCH, 2 * EMBED), jnp.float32),
            jax.ShapeDtypeStruct((BATCH * EMBED,), jnp.float32),
        ),
        scratch_types=[
            pltpu.VMEM((BPW,), jnp.int32),
            pltpu.VMEM((BPW,), jnp.int32),
            pltpu.VMEM((BPW, 2 * EMBED), jnp.float32),
            pltpu.VMEM((BPW * EMBED,), jnp.float32),
            pltpu.SemaphoreType.DMA,
        ],
    )(_gather_body)
    return fn(uix, pix, ut2, pt1)


def _mlp_body(u2, upar, p, f, w1u, w1p, w1f, b1, w2, b2, w3t, b3, o):
    par = (upar[:] == 1)
    u = jnp.where(par, u2[:, EMBED:], u2[:, :EMBED])
    x1 = jnp.dot(u, w1u[:], preferred_element_type=jnp.float32)
    x1 = x1 + jnp.dot(p[:], w1p[:], preferred_element_type=jnp.float32)
    x1 = x1 + jnp.dot(f[:], w1f[:], preferred_element_type=jnp.float32)
    h1 = jnp.maximum(x1 + b1[:], 0.0)
    h2 = jnp.maximum(
        jnp.dot(h1, w2[:], preferred_element_type=jnp.float32) + b2[:], 0.0)
    s = jnp.sum(h2 * w3t[:], axis=1, keepdims=True) + b3[:]
    o[:] = 1.0 / (1.0 + jnp.exp(-s))


def _tc_mlp(u2, upar, p, f, w1u, w1p, w1f, b1, w2, b2, w3t, b3, tile=512):
    grid = BATCH // tile
    full = lambda i: (0, 0)
    return pl.pallas_call(
        _mlp_body,
        grid=(grid,),
        in_specs=[
            pl.BlockSpec((tile, 2 * EMBED), lambda i: (i, 0)),
            pl.BlockSpec((tile, 1), lambda i: (i, 0)),
            pl.BlockSpec((tile, EMBED), lambda i: (i, 0)),
            pl.BlockSpec((tile, FEAT), lambda i: (i, 0)),
            pl.BlockSpec((EMBED, HID), full),
            pl.BlockSpec((EMBED, HID), full),
            pl.BlockSpec((FEAT, HID), full),
            pl.BlockSpec((1, HID), full),
            pl.BlockSpec((HID, HID), full),
            pl.BlockSpec((1, HID), full),
            pl.BlockSpec((1, HID), full),
            pl.BlockSpec((1, 1), full),
        ],
        out_specs=pl.BlockSpec((tile, 1), lambda i: (i, 0)),
        out_shape=jax.ShapeDtypeStruct((BATCH, 1), jnp.float32),
    )(u2, upar, p, f, w1u, w1p, w1f, b1, w2, b2, w3t, b3)


def kernel(user_indices, post_indices, features, user_table, post_table,
           W1, b1, W2, b2, W3, b3):
    ui = user_indices.astype(jnp.int32)
    pi = post_indices.astype(jnp.int32)
    u2, p1 = _sc_gather(ui, pi,
                        user_table.reshape(NROWS // 2, 2 * EMBED),
                        post_table.reshape(NROWS * EMBED))
    o = _tc_mlp(
        u2, (ui & 1).reshape(BATCH, 1), p1.reshape(BATCH, EMBED), features,
        W1[:EMBED], W1[EMBED:2 * EMBED], W1[2 * EMBED:],
        b1.reshape(1, HID), W2, b2.reshape(1, HID),
        W3.reshape(1, HID), b3.reshape(1, 1))
    return o.reshape(BATCH)


# speedup vs baseline: 1.2608x; 1.2546x over previous
import functools

import jax
import jax.numpy as jnp
from jax import lax
from jax.experimental import pallas as pl
from jax.experimental.pallas import tpu as pltpu
from jax.experimental.pallas import tpu_sc as plsc

BATCH = 16384
EMBED = 64
FEAT = 128
HID = 128
NROWS = 1000000

NC = 2
NS = 16
NW = NC * NS
BPW = BATCH // NW


def _gather_body(uix, pix, ut2, pt1, u_out, p_out,
                 uix_v, pix_v, urows_v, prows_v, sem):
    wid = lax.axis_index("s") * NC + lax.axis_index("c")
    base = wid * BPW
    pltpu.sync_copy(uix.at[pl.ds(base, BPW)], uix_v)
    pltpu.sync_copy(pix.at[pl.ds(base, BPW)], pix_v)

    def group(g):
        vu = uix_v[pl.ds(g * 16, 16)]
        vp = pix_v[pl.ds(g * 16, 16)]
        for lane in range(16):
            i = g * 16 + lane
            pltpu.async_copy(
                ut2.at[pl.ds(vu[lane], 1)],
                urows_v.at[pl.ds(i, 1)], sem)
            pltpu.async_copy(
                pt1.at[pl.ds(vp[lane] * EMBED, EMBED)],
                prows_v.at[pl.ds(i * EMBED, EMBED)], sem)

    pl.loop(0, BPW // 16)(group)
    pltpu.make_async_copy(
        ut2.at[pl.ds(0, BPW)], urows_v, sem).wait()
    pltpu.make_async_copy(
        pt1.at[pl.ds(0, BPW * EMBED)], prows_v, sem).wait()
    pltpu.sync_copy(urows_v, u_out.at[pl.ds(base, BPW)])
    pltpu.sync_copy(prows_v, p_out.at[pl.ds(base * EMBED, BPW * EMBED)])


def _sc_gather(uix, pix, ut2, pt1):
    mesh = plsc.VectorSubcoreMesh(core_axis_name="c", subcore_axis_name="s")
    fn = functools.partial(
        pl.kernel,
        mesh=mesh,
        out_type=(
            jax.ShapeDtypeStruct((BATCH, EMBED), jnp.float32),
            jax.ShapeDtypeStruct((BATCH * EMBED,), jnp.float32),
        ),
        scratch_types=[
            pltpu.VMEM((BPW,), jnp.int32),
            pltpu.VMEM((BPW,), jnp.int32),
            pltpu.VMEM((BPW, EMBED), jnp.float32),
            pltpu.VMEM((BPW * EMBED,), jnp.float32),
            pltpu.SemaphoreType.DMA,
        ],
    )(_gather_body)
    return fn(uix, pix, ut2, pt1)


def _mlp_body(u, p, f, w1u, w1p, w1f, b1, w2, b2, w3t, b3, o):
    x1 = jnp.dot(u[:], w1u[:], preferred_element_type=jnp.float32)
    x1 = x1 + jnp.dot(p[:], w1p[:], preferred_element_type=jnp.float32)
    x1 = x1 + jnp.dot(f[:], w1f[:], preferred_element_type=jnp.float32)
    h1 = jnp.maximum(x1 + b1[:], 0.0)
    h2 = jnp.maximum(
        jnp.dot(h1, w2[:], preferred_element_type=jnp.float32) + b2[:], 0.0)
    s = jnp.sum(h2 * w3t[:], axis=1, keepdims=True) + b3[:]
    o[:] = 1.0 / (1.0 + jnp.exp(-s))


def _tc_mlp(u, p, f, w1u, w1p, w1f, b1, w2, b2, w3t, b3, tile=512):
    grid = BATCH // tile
    full = lambda i: (0, 0)
    return pl.pallas_call(
        _mlp_body,
        grid=(grid,),
        in_specs=[
            pl.BlockSpec((tile, EMBED), lambda i: (i, 0)),
            pl.BlockSpec((tile, EMBED), lambda i: (i, 0)),
            pl.BlockSpec((tile, FEAT), lambda i: (i, 0)),
            pl.BlockSpec((EMBED, HID), full),
            pl.BlockSpec((EMBED, HID), full),
            pl.BlockSpec((FEAT, HID), full),
            pl.BlockSpec((1, HID), full),
            pl.BlockSpec((HID, HID), full),
            pl.BlockSpec((1, HID), full),
            pl.BlockSpec((1, HID), full),
            pl.BlockSpec((1, 1), full),
        ],
        out_specs=pl.BlockSpec((tile, 1), lambda i: (i, 0)),
        out_shape=jax.ShapeDtypeStruct((BATCH, 1), jnp.float32),
    )(u, p, f, w1u, w1p, w1f, b1, w2, b2, w3t, b3)


def kernel(user_indices, post_indices, features, user_table, post_table,
           W1, b1, W2, b2, W3, b3):
    ui = user_indices.astype(jnp.int32)
    pi = post_indices.astype(jnp.int32)
    u, p1 = _sc_gather(ui, pi, user_table,
                       post_table.reshape(NROWS * EMBED))
    o = _tc_mlp(
        u, p1.reshape(BATCH, EMBED), features,
        W1[:EMBED], W1[EMBED:2 * EMBED], W1[2 * EMBED:],
        b1.reshape(1, HID), W2, b2.reshape(1, HID),
        W3.reshape(1, HID), b3.reshape(1, 1))
    return o.reshape(BATCH)
